# Initial kernel scaffold; baseline (speedup 1.0000x reference)
import jax
import jax.numpy as jnp
from jax.experimental import pallas as pl


def kernel(src, dst, emb1, emb2):
    raise NotImplementedError("write your pallas kernel here")


# speedup vs baseline: 1.0006x; 1.0006x over previous
import functools

import numpy as np
import jax
import jax.numpy as jnp
from jax.experimental import pallas as pl

N_NODES = 100000
N_EDGES = 3200000
EMB_DIM = 16
EPS = 1e-08

_BLK = 25600


def _edge_w_body(e1_ref, e2_ref, o_ref):
    s = jnp.sum(e1_ref[...] * e2_ref[...], axis=0)
    o_ref[...] = 1.0 / (1.0 + jnp.exp(-s))


def _edge_w(e1t, e2t):
    grid = (N_EDGES // _BLK,)
    return pl.pallas_call(
        _edge_w_body,
        grid=grid,
        in_specs=[
            pl.BlockSpec((EMB_DIM, _BLK), lambda i: (np.int32(0), i)),
            pl.BlockSpec((EMB_DIM, _BLK), lambda i: (np.int32(0), i)),
        ],
        out_specs=pl.BlockSpec((_BLK,), lambda i: (i,)),
        out_shape=jax.ShapeDtypeStruct((N_EDGES,), jnp.float32),
    )(e1t, e2t)


def kernel(src, dst, emb1, emb2):
    src32 = src.astype(jnp.int32)
    dst32 = dst.astype(jnp.int32)
    e1t = emb1.T[:, src32]
    e2t = emb2.T[:, dst32]
    w = _edge_w(e1t, e2t)
    row_sum = jnp.zeros((N_NODES,), jnp.float32).at[src32].add(w)
    w_norm = w / (row_sum[src32] + EPS)
    lin = src.astype(jnp.int64) * N_NODES + dst.astype(jnp.int64)
    order = jnp.argsort(lin)
    lin_s = lin[order]
    w_s = w_norm[order]
    new_group = jnp.concatenate([
        jnp.zeros((1,), dtype=jnp.int32),
        (lin_s[1:] != lin_s[:-1]).astype(jnp.int32),
    ])
    seg = jnp.cumsum(new_group)
    vals = jax.ops.segment_sum(w_s, seg, num_segments=N_EDGES)
    rep_lin = jnp.zeros((N_EDGES,), dtype=lin_s.dtype).at[seg].set(lin_s)
    rep_src = rep_lin // N_NODES
    rep_dst = rep_lin % N_NODES
    idx = jnp.stack([rep_src, rep_dst], axis=0)
    return idx, vals


# speedup vs baseline: 38.8146x; 38.7896x over previous
import functools

import numpy as np
import jax
import jax.numpy as jnp
from jax import lax
from jax.experimental import pallas as pl
from jax.experimental.pallas import tpu as pltpu
from jax.experimental.pallas import tpu_sc as plsc

N = 3200000
V = 100000
EMB = 16
EPS = 1e-08

NC = 2
NS = 16
NW = NC * NS

I32 = jnp.int32
F32 = jnp.float32

WIN = 1024
NWIN = N // WIN
WIN_Q, WIN_R = divmod(NWIN, NW)

KWIN = 512
KNWIN = N // KWIN
KWIN_Q, KWIN_R = divmod(KNWIN, NW)

SSEG = 800000
NROUND = 4
SSUB = SSEG // NS

NB = [2048, 4096, 2048]

RING = 4096
RINGPAD = RING + WIN

_MESH = plsc.VectorSubcoreMesh(core_axis_name="c", subcore_axis_name="s")
_CP = pltpu.CompilerParams(needs_layout_passes=False)

IOTA = lambda: lax.iota(I32, 16)


def _mo(x, m):
    return pl.multiple_of(x, m)


def _wid():
    return lax.axis_index("s") * NC + lax.axis_index("c")


def _full(v):
    return jnp.zeros((16,), I32) + v


def _win_range(wid, q, r):
    start = wid * q + jnp.minimum(wid, r)
    cnt = jnp.where(wid < r, I32(q + 1), I32(q))
    return start.astype(I32), cnt


def _splat_lane(ref16, lane):
    return plsc.load_gather(ref16, [_full(lane)])


def _srl(x, k):
    return lax.shift_right_logical(x, jnp.full(x.shape, k, I32))


def _digit(p, a, b):
    if p == 0:
        return a & 0x7FF
    if p == 1:
        return ((_srl(a, 17) & 0x3F) << 6) | (_srl(a, 11) & 0x3F)
    return ((_srl(b, 22) & 0x3) << 9) | (_srl(a, 23) & 0x1FF)


def _decode_src(a, b):
    return (_srl(a, 17) & 0x7FFF) | ((_srl(b, 22) & 0x3) << 15)


def _k1_body(src_hbm, dst_hbm, e1_hbm, e2_hbm, w_hbm, rs_hbm,
             s2d, d2d, gi, rows1, rows2, w2d, zb, rs_sh, sem):
    wid = _wid()
    cid = lax.axis_index("c")
    sid = lax.axis_index("s")
    nz = 6256 // 16
    def zinit(i, _):
        zb[pl.ds(i * 16, 16)] = jnp.zeros((16,), F32)
        return 0
    lax.fori_loop(I32(0), I32(nz), zinit, 0)

    @pl.when(sid < 15)
    def _():
        pltpu.sync_copy(zb.at[pl.ds(0, 6256)],
                        rs_sh.at[pl.ds(_mo(sid * 6256, 16), 6256)])

    @pl.when(sid == 15)
    def _():
        pltpu.sync_copy(zb.at[pl.ds(0, 6160)],
                        rs_sh.at[pl.ds(_mo(sid * 6256, 16), 6160)])
    plsc.subcore_barrier()

    start, cnt = _win_range(wid, KWIN_Q, KWIN_R)
    nslice = KWIN // 128

    def win(j, _):
        off = _mo((start + j) * KWIN, 128)
        for k in range(nslice):
            pltpu.sync_copy(src_hbm.at[pl.ds(_mo(off + k * 128, 128), 128)], s2d.at[I32(k)])
            pltpu.sync_copy(dst_hbm.at[pl.ds(_mo(off + k * 128, 128), 128)], d2d.at[I32(k)])

        def bidx(src2d):
            def be(e, _):
                nv = plsc.load_gather(src2d, [_srl(_full(e), 7), _full(e) & 127])
                gi[_srl(e, 3), pl.ds((e & 7) * 16, 16)] = nv * 16 + IOTA()
                return 0
            lax.fori_loop(I32(0), I32(KWIN), be, 0)

        bidx(s2d)
        hs = [pltpu.async_copy(e1_hbm.at[gi.at[I32(r)]],
                               rows1.at[pl.ds(r * 128, 128)], sem)
              for r in range(KWIN * 16 // 128)]
        for h in hs:
            h.wait()
        bidx(d2d)
        hs = [pltpu.async_copy(e2_hbm.at[gi.at[I32(r)]],
                               rows2.at[pl.ds(r * 128, 128)], sem)
              for r in range(KWIN * 16 // 128)]
        for h in hs:
            h.wait()

        def dot(v, _):
            base = v * 256
            acc = jnp.zeros((16,), F32)
            ii = IOTA() * 16
            for d in range(16):
                idx = ii + (base + d)
                acc = acc + plsc.load_gather(rows1, [idx]) * plsc.load_gather(rows2, [idx])
            w = 1.0 / (1.0 + jnp.exp(-acc))
            w2d[_srl(v, 3), pl.ds((v & 7) * 16, 16)] = w
            return 0
        lax.fori_loop(I32(0), I32(KWIN // 16), dot, 0)

        for k in range(nslice):
            pltpu.sync_copy(w2d.at[I32(k)], w_hbm.at[pl.ds(_mo(off + k * 128, 128), 128)])
        hs = [pltpu.async_copy(w2d.at[I32(k)], rs_sh.at[s2d.at[I32(k)]], sem, add=True)
              for k in range(nslice)]
        for h in hs:
            h.wait()
        return 0

    lax.fori_loop(I32(0), cnt, win, 0)
    plsc.subcore_barrier()

    @pl.when(sid < 15)
    def _():
        pltpu.sync_copy(rs_sh.at[pl.ds(_mo(sid * 6256, 16), 6256)],
                        zb.at[pl.ds(0, 6256)])
        pltpu.sync_copy(zb.at[pl.ds(0, 6256)],
                        rs_hbm.at[pl.ds(_mo(cid * V + sid * 6256, 16), 6256)])

    @pl.when(sid == 15)
    def _():
        pltpu.sync_copy(rs_sh.at[pl.ds(_mo(sid * 6256, 16), 6160)],
                        zb.at[pl.ds(0, 6160)])
        pltpu.sync_copy(zb.at[pl.ds(0, 6160)],
                        rs_hbm.at[pl.ds(_mo(cid * V + sid * 6256, 16), 6160)])


def _k1(src32, dst32, e1flat, e2flat):
    kern = functools.partial(
        pl.kernel,
        out_type=[jax.ShapeDtypeStruct((N,), F32),
                  jax.ShapeDtypeStruct((NC * V,), F32)],
        mesh=_MESH, compiler_params=_CP,
        scratch_types=[
            pltpu.VMEM((KWIN // 128, 128), I32),
            pltpu.VMEM((KWIN // 128, 128), I32),
            pltpu.VMEM((KWIN * 16 // 128, 128), I32),
            pltpu.VMEM((KWIN * 16,), F32),
            pltpu.VMEM((KWIN * 16,), F32),
            pltpu.VMEM((KWIN // 128, 128), F32),
            pltpu.VMEM((6256,), F32),
            pltpu.VMEM_SHARED((V,), F32),
            pltpu.SemaphoreType.DMA,
        ],
    )(_k1_body)
    return kern(src32, dst32, e1flat, e2flat)


def _h_body(p, nb, in1_hbm, in2_hbm, hist_hbm, abuf, bbuf, hist):
    wid = _wid()
    def zi(i, _):
        hist[pl.ds(i * 16, 16)] = jnp.zeros((16,), I32)
        return 0
    lax.fori_loop(I32(0), I32(nb // 16), zi, 0)

    start, cnt = _win_range(wid, WIN_Q, WIN_R)

    def win(j, _):
        off = _mo((start + j) * WIN, 128)
        pltpu.sync_copy(in1_hbm.at[pl.ds(off, WIN)], abuf)
        pltpu.sync_copy(in2_hbm.at[pl.ds(off, WIN)], bbuf)

        def vr(v, _):
            a = abuf[pl.ds(v * 16, 16)]
            b = bbuf[pl.ds(v * 16, 16)]
            if p == 0:
                d = b & 0x7FF
            else:
                d = _digit(p, a, b)
            cntv, lastm = plsc.scan_count(d)
            base = plsc.load_gather(hist, [d])
            plsc.store_scatter(hist, [d], base + cntv, mask=lastm)
            return 0
        lax.fori_loop(I32(0), I32(WIN // 16), vr, 0)
        return 0

    lax.fori_loop(I32(0), cnt, win, 0)
    pltpu.sync_copy(hist, hist_hbm.at[pl.ds(_mo(wid * nb, 128), nb)])


def _hist(p, in1, in2):
    nb = NB[p]
    body = functools.partial(_h_body, p, nb)
    kern = functools.partial(
        pl.kernel,
        out_type=jax.ShapeDtypeStruct((NW * nb,), I32),
        mesh=_MESH, compiler_params=_CP,
        scratch_types=[
            pltpu.VMEM((WIN,), I32),
            pltpu.VMEM((WIN,), I32),
            pltpu.VMEM((nb,), I32),
        ],
    )(body)
    return kern(in1, in2)


def _offsets(hist, nb):
    hist = hist.reshape(NW, nb)
    flat = hist.T.reshape(-1).astype(I32)
    ex = jnp.concatenate([jnp.zeros((1,), I32), jnp.cumsum(flat)[:-1].astype(I32)])
    return ex.reshape(nb, NW).T.reshape(-1)


def _p_body(p, nb, in1_hbm, in2_hbm, offs_hbm, a_out, b_out,
            abuf, bbuf, av2d, bv2d, pos2d, counter, zb, bounce,
            segA, segB, sem):
    wid = _wid()
    cid = lax.axis_index("c")
    sid = lax.axis_index("s")
    start, cnt = _win_range(wid, WIN_Q, WIN_R)
    nzc = SSUB // 2000

    def zvi(i, _):
        zb[pl.ds(i * 16, 16)] = jnp.zeros((16,), I32)
        return 0
    lax.fori_loop(I32(0), I32(2000 // 16), zvi, 0)

    for r in range(NROUND):
        def zc(i, _):
            pltpu.sync_copy(zb, segA.at[pl.ds(_mo(sid * SSUB + i * 2000, 8), 2000)])
            pltpu.sync_copy(zb, segB.at[pl.ds(_mo(sid * SSUB + i * 2000, 8), 2000)])
            return 0
        lax.fori_loop(I32(0), I32(nzc), zc, 0)
        plsc.subcore_barrier()

        pltpu.sync_copy(offs_hbm.at[pl.ds(_mo(wid * nb, 128), nb)], counter)
        lo = r * SSEG

        def win(j, _):
            off = _mo((start + j) * WIN, 128)
            pltpu.sync_copy(in1_hbm.at[pl.ds(off, WIN)], abuf)
            pltpu.sync_copy(in2_hbm.at[pl.ds(off, WIN)], bbuf)

            def vr(v, _):
                x1 = abuf[pl.ds(v * 16, 16)]
                x2 = bbuf[pl.ds(v * 16, 16)]
                if p == 0:
                    tag = off + v * 16 + IOTA()
                    a = x2 | ((x1 & 0x7FFF) << 17)
                    b = tag | (_srl(x1, 15) << 22)
                    d = x2 & 0x7FF
                else:
                    a, b = x1, x2
                    d = _digit(p, a, b)
                cntv, lastm = plsc.scan_count(d)
                base = plsc.load_gather(counter, [d])
                plsc.store_scatter(counter, [d], base + cntv, mask=lastm)
                pos = base + cntv - 1
                m = (pos >= lo) & (pos < lo + SSEG)
                iv = jnp.where(m, pos - lo, SSEG + IOTA())
                rr = lax.div(v, I32(8))
                cc = lax.rem(v, I32(8)) * 16
                pos2d[rr, pl.ds(cc, 16)] = iv
                av2d[rr, pl.ds(cc, 16)] = a
                bv2d[rr, pl.ds(cc, 16)] = b
                return 0
            lax.fori_loop(I32(0), I32(WIN // 16), vr, 0)

            hs = []
            for k in range(WIN // 128):
                hs.append(pltpu.async_copy(av2d.at[I32(k)], segA.at[pos2d.at[I32(k)]], sem))
                hs.append(pltpu.async_copy(bv2d.at[I32(k)], segB.at[pos2d.at[I32(k)]], sem))
            for h in hs:
                h.wait()
            return 0

        lax.fori_loop(I32(0), cnt, win, 0)
        plsc.subcore_barrier()

        def ec(i, _):
            o = _mo(sid * SSUB + i * 2000, 8)
            pltpu.sync_copy(segA.at[pl.ds(o, 2000)], bounce)
            pltpu.sync_copy(bounce, a_out.at[pl.ds(_mo(cid * N + lo + o, 8), 2000)])
            pltpu.sync_copy(segB.at[pl.ds(o, 2000)], bounce)
            pltpu.sync_copy(bounce, b_out.at[pl.ds(_mo(cid * N + lo + o, 8), 2000)])
            return 0
        lax.fori_loop(I32(0), I32(nzc), ec, 0)
        plsc.subcore_barrier()


def _permute(p, in1, in2, offs):
    nb = NB[p]
    body = functools.partial(_p_body, p, nb)
    kern = functools.partial(
        pl.kernel,
        out_type=[jax.ShapeDtypeStruct((NC * N,), I32),
                  jax.ShapeDtypeStruct((NC * N,), I32)],
        mesh=_MESH, compiler_params=_CP,
        scratch_types=[
            pltpu.VMEM((WIN,), I32),
            pltpu.VMEM((WIN,), I32),
            pltpu.VMEM((WIN // 128, 128), I32),
            pltpu.VMEM((WIN // 128, 128), I32),
            pltpu.VMEM((WIN // 128, 128), I32),
            pltpu.VMEM((nb,), I32),
            pltpu.VMEM((2000,), I32),
            pltpu.VMEM((2000,), I32),
            pltpu.VMEM_SHARED((SSEG + 16,), I32),
            pltpu.VMEM_SHARED((SSEG + 16,), I32),
            pltpu.SemaphoreType.DMA,
        ],
    )(body)
    ao, bo = kern(in1, in2, offs)
    return ao[:N] + ao[N:], bo[:N] + bo[N:]


def _flags_and_scan(abuf, bbuf, wv, rsv, shs, shb, v, prev_a, prev_b,
                    open_c, base_off):
    el = v * 16 + IOTA()
    a = abuf[pl.ds(v * 16, 16)]
    b = bbuf[pl.ds(v * 16, 16)]
    elm1 = jnp.maximum(el - 1, 0)
    ap = plsc.load_gather(abuf, [elm1])
    bp = plsc.load_gather(bbuf, [elm1])
    first = el == 0
    ap = jnp.where(first, _full(prev_a), ap)
    bp = jnp.where(first, _full(prev_b), bp)
    ge = base_off + el
    flag = ((a != ap) | (_srl(b, 22) != _srl(bp, 22))).astype(I32)
    flag = jnp.where(ge == 0, 0, flag)

    wn = wv / (rsv + EPS)

    fc = plsc.cumsum(flag)
    s = wn
    bb = flag
    for k in (1, 2, 4, 8):
        shs[...] = s
        shb[...] = bb
        gidx = jnp.maximum(IOTA() - k, 0)
        shv = plsc.load_gather(shs, [gidx])
        bhv = plsc.load_gather(shb, [gidx])
        ok = IOTA() >= k
        s = s + jnp.where(ok & (bb == 0), shv, 0.0)
        bb = bb | jnp.where(ok, bhv, 0)
    s = s + jnp.where(fc == 0, open_c, 0.0)

    an = plsc.load_gather(abuf, [el + 1])
    bn = plsc.load_gather(bbuf, [el + 1])
    flagn = ((a != an) | (_srl(b, 22) != _srl(bn, 22)))
    is_tail = flagn | (ge == N - 1)
    return dict(a=a, b=b, el=el, flag=flag, fc=fc, s=s, is_tail=is_tail)


def _stage_fwin(in1_hbm, in2_hbm, w_hbm, rs_hbm, abuf, bbuf, t2d, s2d,
                w2d, rs2d, off, sem):
    pltpu.sync_copy(in1_hbm.at[pl.ds(_mo(off, 128), WIN + 16)], abuf)
    pltpu.sync_copy(in2_hbm.at[pl.ds(_mo(off, 128), WIN + 16)], bbuf)

    def dv(v, _):
        a = abuf[pl.ds(v * 16, 16)]
        b = bbuf[pl.ds(v * 16, 16)]
        rr = lax.div(v, I32(8))
        cc = lax.rem(v, I32(8)) * 16
        t2d[rr, pl.ds(cc, 16)] = b & 0x3FFFFF
        s2d[rr, pl.ds(cc, 16)] = _decode_src(a, b)
        return 0
    lax.fori_loop(I32(0), I32(WIN // 16), dv, 0)
    hs = []
    for k in range(WIN // 128):
        hs.append(pltpu.async_copy(w_hbm.at[t2d.at[I32(k)]], w2d.at[I32(k)], sem))
        hs.append(pltpu.async_copy(rs_hbm.at[s2d.at[I32(k)]], rs2d.at[I32(k)], sem))
    for h in hs:
        h.wait()


def _f1_body(in1_hbm, in2_hbm, w_hbm, rs_hbm, fc_hbm, ts_hbm,
             abuf, bbuf, t2d, s2d, w2d, rs2d, shs, shb, mb, mbf, sem):
    wid = _wid()
    start, cnt = _win_range(wid, WIN_Q, WIN_R)
    base0 = start * WIN

    @pl.when(wid > 0)
    def _():
        pltpu.sync_copy(in1_hbm.at[pl.ds(_mo(base0 - 128, 128), 128)], abuf.at[pl.ds(0, 128)])
        pltpu.sync_copy(in2_hbm.at[pl.ds(_mo(base0 - 128, 128), 128)], bbuf.at[pl.ds(0, 128)])
    pa0 = jnp.sum(jnp.where(IOTA() == 15, abuf[pl.ds(112, 16)], 0), dtype=I32)
    pb0 = jnp.sum(jnp.where(IOTA() == 15, bbuf[pl.ds(112, 16)], 0), dtype=I32)

    def win(j, carry):
        prev_a, prev_b, open_c, nflag = carry
        off = base0 + j * WIN
        _stage_fwin(in1_hbm, in2_hbm, w_hbm, rs_hbm, abuf, bbuf, t2d, s2d,
                    w2d, rs2d, off, sem)

        def vr(v, vc):
            p_a, p_b, op, nf = vc
            rr = lax.div(v, I32(8))
            cc = lax.rem(v, I32(8)) * 16
            wv = w2d[rr, pl.ds(cc, 16)]
            rsv = rs2d[rr, pl.ds(cc, 16)]
            r = _flags_and_scan(abuf, bbuf, wv, rsv, shs, shb, v, p_a, p_b,
                                op, off)
            s15 = jnp.sum(jnp.where(IOTA() == 15, r["s"], 0.0))
            a15 = jnp.sum(jnp.where(IOTA() == 15, r["a"], 0), dtype=I32)
            b15 = jnp.sum(jnp.where(IOTA() == 15, r["b"], 0), dtype=I32)
            nf2 = nf + jnp.sum(r["flag"], dtype=I32)
            return (a15, b15, s15, nf2)
        p_a, p_b, op, nf = lax.fori_loop(I32(0), I32(WIN // 16), vr,
                                         (prev_a, prev_b, open_c, nflag))
        return (p_a, p_b, op, nf)

    _, _, open_f, nflag = lax.fori_loop(
        I32(0), cnt, win, (pa0, pb0, jnp.float32(0.0), I32(0)))

    mb[...] = jnp.where(IOTA() == 0, _full(nflag), 0)
    pltpu.sync_copy(mb, fc_hbm.at[pl.ds(_mo(wid * 16, 16), 16)])
    mbf[...] = jnp.where(IOTA() == 0, jnp.zeros((16,), F32) + open_f, 0.0)
    pltpu.sync_copy(mbf, ts_hbm.at[pl.ds(_mo(wid * 16, 16), 16)])


def _f1(in1, in2, w, rs):
    kern = functools.partial(
        pl.kernel,
        out_type=[jax.ShapeDtypeStruct((NW * 16,), I32),
                  jax.ShapeDtypeStruct((NW * 16,), F32)],
        mesh=_MESH, compiler_params=_CP,
        scratch_types=[
            pltpu.VMEM((WIN + 16,), I32),
            pltpu.VMEM((WIN + 16,), I32),
            pltpu.VMEM((WIN // 128, 128), I32),
            pltpu.VMEM((WIN // 128, 128), I32),
            pltpu.VMEM((WIN // 128, 128), F32),
            pltpu.VMEM((WIN // 128, 128), F32),
            pltpu.VMEM((16,), F32),
            pltpu.VMEM((16,), I32),
            pltpu.VMEM((16,), I32),
            pltpu.VMEM((16,), F32),
            pltpu.SemaphoreType.DMA,
        ],
    )(_f1_body)
    return kern(in1, in2, w, rs)


def _f2_body(in1_hbm, in2_hbm, w_hbm, rs_hbm, auxi_hbm, auxf_hbm,
             vals_hbm, srcO_hbm, dstO_hbm, bv_hbm, bs_hbm, bd_hbm,
             hv_hbm, hs_hbm, hd_hbm, hm_hbm, bm_hbm,
             abuf, bbuf, t2d, s2d, w2d, rs2d, shs, shb,
             ringv, rings, ringd, headv, heads, headd, hmk, mb, mbf, sem):
    wid = _wid()
    start, cnt = _win_range(wid, WIN_Q, WIN_R)
    base0 = start * WIN

    pltpu.sync_copy(auxi_hbm.at[pl.ds(_mo(wid * 16, 16), 16)], mb)
    sb = jnp.max(mb[...])
    pltpu.sync_copy(auxf_hbm.at[pl.ds(_mo(wid * 16, 16), 16)], mbf)
    inc0 = jnp.max(mbf[...])
    fb0 = (sb + 7) & (-8)
    hmk[...] = jnp.zeros((16,), I32)

    @pl.when(wid > 0)
    def _():
        pltpu.sync_copy(in1_hbm.at[pl.ds(_mo(base0 - 128, 128), 128)], abuf.at[pl.ds(0, 128)])
        pltpu.sync_copy(in2_hbm.at[pl.ds(_mo(base0 - 128, 128), 128)], bbuf.at[pl.ds(0, 128)])
    pa0 = jnp.sum(jnp.where(IOTA() == 15, abuf[pl.ds(112, 16)], 0), dtype=I32)
    pb0 = jnp.sum(jnp.where(IOTA() == 15, bbuf[pl.ds(112, 16)], 0), dtype=I32)

    def win(j, carry):
        prev_a, prev_b, open_c, seg_c, s1, fbase = carry
        off = base0 + j * WIN
        _stage_fwin(in1_hbm, in2_hbm, w_hbm, rs_hbm, abuf, bbuf, t2d, s2d,
                    w2d, rs2d, off, sem)

        def vr(v, vc):
            p_a, p_b, op, sc, ss1 = vc
            rr = lax.div(v, I32(8))
            cc = lax.rem(v, I32(8)) * 16
            wv = w2d[rr, pl.ds(cc, 16)]
            rsv = rs2d[rr, pl.ds(cc, 16)]
            r = _flags_and_scan(abuf, bbuf, wv, rsv, shs, shb, v, p_a, p_b,
                                op, off)
            seg = sc + r["fc"]
            is_tail = r["is_tail"]
            srcv = s2d[rr, pl.ds(cc, 16)]
            dstv = r["a"] & 0x1FFFF
            idx0 = seg & (RING - 1)
            plsc.store_scatter(ringv, [idx0], r["s"], mask=is_tail)
            plsc.store_scatter(rings, [idx0], srcv, mask=is_tail)
            plsc.store_scatter(ringd, [idx0], dstv, mask=is_tail)
            mwrap = is_tail & (idx0 < WIN)
            idx1 = jnp.where(idx0 < WIN, idx0 + RING, 0)
            plsc.store_scatter(ringv, [idx1], r["s"], mask=mwrap)
            plsc.store_scatter(rings, [idx1], srcv, mask=mwrap)
            plsc.store_scatter(ringd, [idx1], dstv, mask=mwrap)
            mh = is_tail & (seg < fb0)
            hix = jnp.clip(seg - sb, 0, 15)
            plsc.store_scatter(headv, [hix], r["s"], mask=mh)
            plsc.store_scatter(heads, [hix], srcv, mask=mh)
            plsc.store_scatter(headd, [hix], dstv, mask=mh)
            plsc.store_scatter(hmk, [hix], _full(1), mask=mh)
            s15 = jnp.sum(jnp.where(IOTA() == 15, r["s"], 0.0))
            a15 = jnp.sum(jnp.where(IOTA() == 15, r["a"], 0), dtype=I32)
            b15 = jnp.sum(jnp.where(IOTA() == 15, r["b"], 0), dtype=I32)
            sc2 = sc + jnp.sum(r["flag"], dtype=I32)
            ss1b = jnp.maximum(ss1, jnp.max(jnp.where(is_tail, seg, -1)))
            return (a15, b15, s15, sc2, ss1b)

        p_a, p_b, op, sc, ss1 = lax.fori_loop(
            I32(0), I32(WIN // 16), vr, (prev_a, prev_b, open_c, seg_c, s1))

        do_flush = fbase + (WIN - 1) <= ss1

        @pl.when(do_flush)
        def _():
            o = _mo(fbase & (RING - 1), 8)
            fb8 = _mo(fbase, 8)
            pltpu.sync_copy(ringv.at[pl.ds(o, WIN)], vals_hbm.at[pl.ds(fb8, WIN)])
            pltpu.sync_copy(rings.at[pl.ds(o, WIN)], srcO_hbm.at[pl.ds(fb8, WIN)])
            pltpu.sync_copy(ringd.at[pl.ds(o, WIN)], dstO_hbm.at[pl.ds(fb8, WIN)])
        fbase = jnp.where(do_flush, fbase + WIN, fbase)
        return (p_a, p_b, op, sc, ss1, fbase)

    init = (pa0, pb0, jnp.float32(0.0), sb, sb - 1, fb0)
    _, _, _, _, s1f, fbf = lax.fori_loop(I32(0), cnt, win, init)

    o = _mo(fbf & (RING - 1), 8)
    pltpu.sync_copy(ringv.at[pl.ds(o, WIN)], bv_hbm.at[pl.ds(_mo(wid * WIN, 128), WIN)])
    pltpu.sync_copy(rings.at[pl.ds(o, WIN)], bs_hbm.at[pl.ds(_mo(wid * WIN, 128), WIN)])
    pltpu.sync_copy(ringd.at[pl.ds(o, WIN)], bd_hbm.at[pl.ds(_mo(wid * WIN, 128), WIN)])
    pltpu.sync_copy(headv, hv_hbm.at[pl.ds(_mo(wid * 16, 16), 16)])
    pltpu.sync_copy(heads, hs_hbm.at[pl.ds(_mo(wid * 16, 16), 16)])
    pltpu.sync_copy(headd, hd_hbm.at[pl.ds(_mo(wid * 16, 16), 16)])
    pltpu.sync_copy(hmk, hm_hbm.at[pl.ds(_mo(wid * 16, 16), 16)])
    meta = jnp.where(IOTA() == 0, _full(fbf),
                     jnp.where(IOTA() == 1, _full(s1f + 1 - fbf),
                               jnp.where(IOTA() == 2, _full(sb),
                                         jnp.where(IOTA() == 3, _full(fb0 - sb),
                                                   _full(0)))))
    mb[...] = meta
    pltpu.sync_copy(mb, bm_hbm.at[pl.ds(_mo(wid * 16, 16), 16)])


def _f2(in1, in2, w, rs, aux_i, aux_f):
    kern = functools.partial(
        pl.kernel,
        out_type=[jax.ShapeDtypeStruct((N + WIN,), F32),
                  jax.ShapeDtypeStruct((N + WIN,), I32),
                  jax.ShapeDtypeStruct((N + WIN,), I32),
                  jax.ShapeDtypeStruct((NW * WIN,), F32),
                  jax.ShapeDtypeStruct((NW * WIN,), I32),
                  jax.ShapeDtypeStruct((NW * WIN,), I32),
                  jax.ShapeDtypeStruct((NW * 16,), F32),
                  jax.ShapeDtypeStruct((NW * 16,), I32),
                  jax.ShapeDtypeStruct((NW * 16,), I32),
                  jax.ShapeDtypeStruct((NW * 16,), I32),
                  jax.ShapeDtypeStruct((NW * 16,), I32)],
        mesh=_MESH, compiler_params=_CP,
        scratch_types=[
            pltpu.VMEM((WIN + 16,), I32),
            pltpu.VMEM((WIN + 16,), I32),
            pltpu.VMEM((WIN // 128, 128), I32),
            pltpu.VMEM((WIN // 128, 128), I32),
            pltpu.VMEM((WIN // 128, 128), F32),
            pltpu.VMEM((WIN // 128, 128), F32),
            pltpu.VMEM((16,), F32),
            pltpu.VMEM((16,), I32),
            pltpu.VMEM((RINGPAD,), F32),
            pltpu.VMEM((RINGPAD,), I32),
            pltpu.VMEM((RINGPAD,), I32),
            pltpu.VMEM((16,), F32),
            pltpu.VMEM((16,), I32),
            pltpu.VMEM((16,), I32),
            pltpu.VMEM((16,), I32),
            pltpu.VMEM((16,), I32),
            pltpu.VMEM((16,), F32),
            pltpu.SemaphoreType.DMA,
        ],
    )(_f2_body)
    return kern(in1, in2, w, rs, aux_i, aux_f)


def kernel(src, dst, emb1, emb2):
    src32 = src.astype(I32)
    dst32 = dst.astype(I32)
    e1flat = emb1.reshape(-1)
    e2flat = emb2.reshape(-1)

    w, rs2 = _k1(src32, dst32, e1flat, e2flat)
    rs = rs2[:V] + rs2[V:]

    a, b = src32, dst32
    for p in range(3):
        h = _hist(p, a, b)
        offs = _offsets(h, NB[p])
        a, b = _permute(p, a, b, offs)

    pad = jnp.zeros((128,), I32)
    a = jnp.concatenate([a, pad])
    b = jnp.concatenate([b, pad])

    fci, tsf = _f1(a, b, w, rs)
    fc = fci.reshape(NW, 16)[:, 0]
    ts = tsf.reshape(NW, 16)[:, 0]
    seg_base = jnp.concatenate(
        [jnp.zeros((1,), I32), jnp.cumsum(fc)[:-1].astype(I32)])
    U = (jnp.sum(fc) + 1).astype(I32)
    inc = []
    carry = jnp.float32(0.0)
    for t in range(NW):
        inc.append(carry)
        carry = ts[t] + jnp.where(fc[t] == 0, carry, jnp.float32(0.0))
    incoming = jnp.stack(inc).astype(F32)
    aux_i = jnp.broadcast_to(seg_base[:, None], (NW, 16)).astype(I32).reshape(-1)
    aux_f = jnp.broadcast_to(incoming[:, None], (NW, 16)).astype(F32).reshape(-1)

    valsR, srcR, dstR, bv, bs, bd, hv, hsv, hd, hm, bm = _f2(
        a, b, w, rs, aux_i, aux_f)
    hm = hm.reshape(NW, 16)
    bv = bv.reshape(NW, WIN)
    bs = bs.reshape(NW, WIN)
    bd = bd.reshape(NW, WIN)
    hv = hv.reshape(NW, 16)
    hsv = hsv.reshape(NW, 16)
    hd = hd.reshape(NW, 16)
    bm = bm.reshape(NW, 16)

    oob = I32(N + WIN + 7)
    ar = jnp.arange(WIN, dtype=I32)
    tpos = bm[:, 0:1] + ar[None, :]
    tmask = ar[None, :] < bm[:, 1:2]
    tgt = jnp.where(tmask, tpos, oob).reshape(-1)
    valsR = valsR.at[tgt].set(bv.reshape(-1), mode="drop")
    srcR = srcR.at[tgt].set(bs.reshape(-1), mode="drop")
    dstR = dstR.at[tgt].set(bd.reshape(-1), mode="drop")

    ar16 = jnp.arange(16, dtype=I32)
    hpos = bm[:, 2:3] + ar16[None, :]
    hmask = (ar16[None, :] < bm[:, 3:4]) & (hm > 0)
    htgt = jnp.where(hmask, hpos, oob).reshape(-1)
    valsR = valsR.at[htgt].set(hv.reshape(-1), mode="drop")
    srcR = srcR.at[htgt].set(hsv.reshape(-1), mode="drop")
    dstR = dstR.at[htgt].set(hd.reshape(-1), mode="drop")

    ii = jnp.arange(N, dtype=I32)
    ok = ii < U
    vals = jnp.where(ok, valsR[:N], jnp.float32(0.0)).astype(F32)
    srcO = jnp.where(ok, srcR[:N], 0)
    dstO = jnp.where(ok, dstR[:N], 0)
    idx = jnp.stack([srcO.astype(jnp.int64), dstO.astype(jnp.int64)], axis=0)
    return idx, vals


# speedup vs baseline: 47.3527x; 1.2200x over previous
import functools

import numpy as np
import jax
import jax.numpy as jnp
from jax import lax
from jax.experimental import pallas as pl
from jax.experimental.pallas import tpu as pltpu
from jax.experimental.pallas import tpu_sc as plsc

N = 3200000
V = 100000
EMB = 16
EPS = 1e-08

NC = 2
NS = 16
NW = NC * NS

I32 = jnp.int32
F32 = jnp.float32

WIN = 1024
NWIN = N // WIN
WIN_Q, WIN_R = divmod(NWIN, NW)

KWIN = 512
KNWIN = N // KWIN
KWIN_Q, KWIN_R = divmod(KNWIN, NW)

SSEG = 800000
NROUND = 4
SSUB = SSEG // NS

NB = [2048, 4096, 2048]

RING = 4096
RINGPAD = RING + WIN

_MESH = plsc.VectorSubcoreMesh(core_axis_name="c", subcore_axis_name="s")
_CP = pltpu.CompilerParams(needs_layout_passes=False)

IOTA = lambda: lax.iota(I32, 16)


def _mo(x, m):
    return pl.multiple_of(x, m)


def _wid():
    return lax.axis_index("s") * NC + lax.axis_index("c")


def _full(v):
    return jnp.zeros((16,), I32) + v


def _win_range(wid, q, r):
    start = wid * q + jnp.minimum(wid, r)
    cnt = jnp.where(wid < r, I32(q + 1), I32(q))
    return start.astype(I32), cnt


def _splat_lane(ref16, lane):
    return plsc.load_gather(ref16, [_full(lane)])


def _srl(x, k):
    return lax.shift_right_logical(x, jnp.full(x.shape, k, I32))


def _digit(p, a, b):
    if p == 0:
        return a & 0x7FF
    if p == 1:
        return ((_srl(a, 17) & 0x3F) << 6) | (_srl(a, 11) & 0x3F)
    return ((_srl(b, 22) & 0x3) << 9) | (_srl(a, 23) & 0x1FF)


def _decode_src(a, b):
    return (_srl(a, 17) & 0x7FFF) | ((_srl(b, 22) & 0x3) << 15)


def _k1_body(src_hbm, dst_hbm, e1_hbm, e2_hbm, w_hbm, rs_hbm,
             s2d, d2d, r1, r2, w2d, zb, rs_sh, sem):
    wid = _wid()
    cid = lax.axis_index("c")
    sid = lax.axis_index("s")
    nz = 6256 // 16
    def zinit(i, _):
        zb[pl.ds(i * 16, 16)] = jnp.zeros((16,), F32)
        return 0
    lax.fori_loop(I32(0), I32(nz), zinit, 0)

    @pl.when(sid < 15)
    def _():
        pltpu.sync_copy(zb.at[pl.ds(0, 6256)],
                        rs_sh.at[pl.ds(_mo(sid * 6256, 16), 6256)])

    @pl.when(sid == 15)
    def _():
        pltpu.sync_copy(zb.at[pl.ds(0, 6160)],
                        rs_sh.at[pl.ds(_mo(sid * 6256, 16), 6160)])
    plsc.subcore_barrier()

    start, cnt = _win_range(wid, KWIN_Q, KWIN_R)
    nslice = KWIN // 128

    def win(j, _):
        off = _mo((start + j) * KWIN, 128)
        for k in range(nslice):
            pltpu.sync_copy(src_hbm.at[pl.ds(_mo(off + k * 128, 128), 128)], s2d.at[I32(k)])
            pltpu.sync_copy(dst_hbm.at[pl.ds(_mo(off + k * 128, 128), 128)], d2d.at[I32(k)])

        for q in range(nslice):
            h1 = pltpu.async_copy(e1_hbm.at[s2d.at[I32(q)]], r1, sem)
            h2 = pltpu.async_copy(e2_hbm.at[d2d.at[I32(q)]], r2, sem)
            h1.wait()
            h2.wait()

            def dot(v, _):
                acc = jnp.zeros((16,), F32)
                ev = v * 16 + IOTA()
                for d in range(16):
                    dd = _full(d)
                    acc = acc + (plsc.load_gather(r1, [ev, dd])
                                 * plsc.load_gather(r2, [ev, dd]))
                w = 1.0 / (1.0 + jnp.exp(-acc))
                w2d[I32(q), pl.ds(v * 16, 16)] = w
                return 0
            lax.fori_loop(I32(0), I32(8), dot, 0)

        for k in range(nslice):
            pltpu.sync_copy(w2d.at[I32(k)], w_hbm.at[pl.ds(_mo(off + k * 128, 128), 128)])
        hs = [pltpu.async_copy(w2d.at[I32(k)], rs_sh.at[s2d.at[I32(k)]], sem, add=True)
              for k in range(nslice)]
        for h in hs:
            h.wait()
        return 0

    lax.fori_loop(I32(0), cnt, win, 0)
    plsc.subcore_barrier()

    @pl.when(sid < 15)
    def _():
        pltpu.sync_copy(rs_sh.at[pl.ds(_mo(sid * 6256, 16), 6256)],
                        zb.at[pl.ds(0, 6256)])
        pltpu.sync_copy(zb.at[pl.ds(0, 6256)],
                        rs_hbm.at[pl.ds(_mo(cid * V + sid * 6256, 16), 6256)])

    @pl.when(sid == 15)
    def _():
        pltpu.sync_copy(rs_sh.at[pl.ds(_mo(sid * 6256, 16), 6160)],
                        zb.at[pl.ds(0, 6160)])
        pltpu.sync_copy(zb.at[pl.ds(0, 6160)],
                        rs_hbm.at[pl.ds(_mo(cid * V + sid * 6256, 16), 6160)])


def _k1(src32, dst32, e1pad, e2pad):
    kern = functools.partial(
        pl.kernel,
        out_type=[jax.ShapeDtypeStruct((N,), F32),
                  jax.ShapeDtypeStruct((NC * V,), F32)],
        mesh=_MESH, compiler_params=_CP,
        scratch_types=[
            pltpu.VMEM((KWIN // 128, 128), I32),
            pltpu.VMEM((KWIN // 128, 128), I32),
            pltpu.VMEM((128, 128), F32),
            pltpu.VMEM((128, 128), F32),
            pltpu.VMEM((KWIN // 128, 128), F32),
            pltpu.VMEM((6256,), F32),
            pltpu.VMEM_SHARED((V,), F32),
            pltpu.SemaphoreType.DMA,
        ],
    )(_k1_body)
    return kern(src32, dst32, e1pad, e2pad)


def _h_body(p, nb, in1_hbm, in2_hbm, hist_hbm, abuf, bbuf, hist):
    wid = _wid()
    def zi(i, _):
        hist[pl.ds(i * 16, 16)] = jnp.zeros((16,), I32)
        return 0
    lax.fori_loop(I32(0), I32(nb // 16), zi, 0)

    start, cnt = _win_range(wid, WIN_Q, WIN_R)

    def win(j, _):
        off = _mo((start + j) * WIN, 128)
        pltpu.sync_copy(in1_hbm.at[pl.ds(off, WIN)], abuf)
        pltpu.sync_copy(in2_hbm.at[pl.ds(off, WIN)], bbuf)

        def vr(v, _):
            a = abuf[pl.ds(v * 16, 16)]
            b = bbuf[pl.ds(v * 16, 16)]
            if p == 0:
                d = b & 0x7FF
            else:
                d = _digit(p, a, b)
            cntv, lastm = plsc.scan_count(d)
            base = plsc.load_gather(hist, [d])
            plsc.store_scatter(hist, [d], base + cntv, mask=lastm)
            return 0
        lax.fori_loop(I32(0), I32(WIN // 16), vr, 0)
        return 0

    lax.fori_loop(I32(0), cnt, win, 0)
    pltpu.sync_copy(hist, hist_hbm.at[pl.ds(_mo(wid * nb, 128), nb)])


def _hist(p, in1, in2):
    nb = NB[p]
    body = functools.partial(_h_body, p, nb)
    kern = functools.partial(
        pl.kernel,
        out_type=jax.ShapeDtypeStruct((NW * nb,), I32),
        mesh=_MESH, compiler_params=_CP,
        scratch_types=[
            pltpu.VMEM((WIN,), I32),
            pltpu.VMEM((WIN,), I32),
            pltpu.VMEM((nb,), I32),
        ],
    )(body)
    return kern(in1, in2)


def _offsets(hist, nb):
    hist = hist.reshape(NW, nb)
    flat = hist.T.reshape(-1).astype(I32)
    ex = jnp.concatenate([jnp.zeros((1,), I32), jnp.cumsum(flat)[:-1].astype(I32)])
    return ex.reshape(nb, NW).T.reshape(-1)


def _p_body(p, nb, in1_hbm, in2_hbm, offs_hbm, a_out, b_out,
            abuf, bbuf, av2d, bv2d, pos2d, counter, zb, bounce,
            segA, segB, sem):
    wid = _wid()
    cid = lax.axis_index("c")
    sid = lax.axis_index("s")
    start, cnt = _win_range(wid, WIN_Q, WIN_R)
    nzc = SSUB // 2000

    def zvi(i, _):
        zb[pl.ds(i * 16, 16)] = jnp.zeros((16,), I32)
        return 0
    lax.fori_loop(I32(0), I32(2000 // 16), zvi, 0)

    for r in range(NROUND):
        def zc(i, _):
            pltpu.sync_copy(zb, segA.at[pl.ds(_mo(sid * SSUB + i * 2000, 8), 2000)])
            pltpu.sync_copy(zb, segB.at[pl.ds(_mo(sid * SSUB + i * 2000, 8), 2000)])
            return 0
        lax.fori_loop(I32(0), I32(nzc), zc, 0)
        plsc.subcore_barrier()

        pltpu.sync_copy(offs_hbm.at[pl.ds(_mo(wid * nb, 128), nb)], counter)
        lo = r * SSEG

        def win(j, _):
            off = _mo((start + j) * WIN, 128)
            pltpu.sync_copy(in1_hbm.at[pl.ds(off, WIN)], abuf)
            pltpu.sync_copy(in2_hbm.at[pl.ds(off, WIN)], bbuf)

            def vr(v, _):
                x1 = abuf[pl.ds(v * 16, 16)]
                x2 = bbuf[pl.ds(v * 16, 16)]
                if p == 0:
                    tag = off + v * 16 + IOTA()
                    a = x2 | ((x1 & 0x7FFF) << 17)
                    b = tag | (_srl(x1, 15) << 22)
                    d = x2 & 0x7FF
                else:
                    a, b = x1, x2
                    d = _digit(p, a, b)
                cntv, lastm = plsc.scan_count(d)
                base = plsc.load_gather(counter, [d])
                plsc.store_scatter(counter, [d], base + cntv, mask=lastm)
                pos = base + cntv - 1
                m = (pos >= lo) & (pos < lo + SSEG)
                iv = jnp.where(m, pos - lo, SSEG + IOTA())
                rr = lax.div(v, I32(8))
                cc = lax.rem(v, I32(8)) * 16
                pos2d[rr, pl.ds(cc, 16)] = iv
                av2d[rr, pl.ds(cc, 16)] = a
                bv2d[rr, pl.ds(cc, 16)] = b
                return 0
            lax.fori_loop(I32(0), I32(WIN // 16), vr, 0)

            hs = []
            for k in range(WIN // 128):
                hs.append(pltpu.async_copy(av2d.at[I32(k)], segA.at[pos2d.at[I32(k)]], sem))
                hs.append(pltpu.async_copy(bv2d.at[I32(k)], segB.at[pos2d.at[I32(k)]], sem))
            for h in hs:
                h.wait()
            return 0

        lax.fori_loop(I32(0), cnt, win, 0)
        plsc.subcore_barrier()

        def ec(i, _):
            o = _mo(sid * SSUB + i * 2000, 8)
            pltpu.sync_copy(segA.at[pl.ds(o, 2000)], bounce)
            pltpu.sync_copy(bounce, a_out.at[pl.ds(_mo(cid * N + lo + o, 8), 2000)])
            pltpu.sync_copy(segB.at[pl.ds(o, 2000)], bounce)
            pltpu.sync_copy(bounce, b_out.at[pl.ds(_mo(cid * N + lo + o, 8), 2000)])
            return 0
        lax.fori_loop(I32(0), I32(nzc), ec, 0)
        plsc.subcore_barrier()


def _permute(p, in1, in2, offs):
    nb = NB[p]
    body = functools.partial(_p_body, p, nb)
    kern = functools.partial(
        pl.kernel,
        out_type=[jax.ShapeDtypeStruct((NC * N,), I32),
                  jax.ShapeDtypeStruct((NC * N,), I32)],
        mesh=_MESH, compiler_params=_CP,
        scratch_types=[
            pltpu.VMEM((WIN,), I32),
            pltpu.VMEM((WIN,), I32),
            pltpu.VMEM((WIN // 128, 128), I32),
            pltpu.VMEM((WIN // 128, 128), I32),
            pltpu.VMEM((WIN // 128, 128), I32),
            pltpu.VMEM((nb,), I32),
            pltpu.VMEM((2000,), I32),
            pltpu.VMEM((2000,), I32),
            pltpu.VMEM_SHARED((SSEG + 16,), I32),
            pltpu.VMEM_SHARED((SSEG + 16,), I32),
            pltpu.SemaphoreType.DMA,
        ],
    )(body)
    ao, bo = kern(in1, in2, offs)
    return ao[:N] + ao[N:], bo[:N] + bo[N:]


def _flags_and_scan(abuf, bbuf, wv, rsv, shs, shb, v, prev_a, prev_b,
                    open_c, base_off):
    el = v * 16 + IOTA()
    a = abuf[pl.ds(v * 16, 16)]
    b = bbuf[pl.ds(v * 16, 16)]
    elm1 = jnp.maximum(el - 1, 0)
    ap = plsc.load_gather(abuf, [elm1])
    bp = plsc.load_gather(bbuf, [elm1])
    first = el == 0
    ap = jnp.where(first, _full(prev_a), ap)
    bp = jnp.where(first, _full(prev_b), bp)
    ge = base_off + el
    flag = ((a != ap) | (_srl(b, 22) != _srl(bp, 22))).astype(I32)
    flag = jnp.where(ge == 0, 0, flag)

    wn = wv / (rsv + EPS)

    fc = plsc.cumsum(flag)
    s = wn
    bb = flag
    for k in (1, 2, 4, 8):
        shs[...] = s
        shb[...] = bb
        gidx = jnp.maximum(IOTA() - k, 0)
        shv = plsc.load_gather(shs, [gidx])
        bhv = plsc.load_gather(shb, [gidx])
        ok = IOTA() >= k
        s = s + jnp.where(ok & (bb == 0), shv, 0.0)
        bb = bb | jnp.where(ok, bhv, 0)
    s = s + jnp.where(fc == 0, open_c, 0.0)

    an = plsc.load_gather(abuf, [el + 1])
    bn = plsc.load_gather(bbuf, [el + 1])
    flagn = ((a != an) | (_srl(b, 22) != _srl(bn, 22)))
    is_tail = flagn | (ge == N - 1)
    return dict(a=a, b=b, el=el, flag=flag, fc=fc, s=s, is_tail=is_tail)


def _stage_fwin(in1_hbm, in2_hbm, w_hbm, rs_hbm, abuf, bbuf, t2d, s2d,
                w2d, rs2d, off, sem):
    pltpu.sync_copy(in1_hbm.at[pl.ds(_mo(off, 128), WIN + 16)], abuf)
    pltpu.sync_copy(in2_hbm.at[pl.ds(_mo(off, 128), WIN + 16)], bbuf)

    def dv(v, _):
        a = abuf[pl.ds(v * 16, 16)]
        b = bbuf[pl.ds(v * 16, 16)]
        rr = lax.div(v, I32(8))
        cc = lax.rem(v, I32(8)) * 16
        t2d[rr, pl.ds(cc, 16)] = b & 0x3FFFFF
        s2d[rr, pl.ds(cc, 16)] = _decode_src(a, b)
        return 0
    lax.fori_loop(I32(0), I32(WIN // 16), dv, 0)
    hs = []
    for k in range(WIN // 128):
        hs.append(pltpu.async_copy(w_hbm.at[t2d.at[I32(k)]], w2d.at[I32(k)], sem))
        hs.append(pltpu.async_copy(rs_hbm.at[s2d.at[I32(k)]], rs2d.at[I32(k)], sem))
    for h in hs:
        h.wait()


def _f1_body(in1_hbm, in2_hbm, w_hbm, rs_hbm, fc_hbm, ts_hbm,
             abuf, bbuf, t2d, s2d, w2d, rs2d, shs, shb, mb, mbf, sem):
    wid = _wid()
    start, cnt = _win_range(wid, WIN_Q, WIN_R)
    base0 = start * WIN

    @pl.when(wid > 0)
    def _():
        pltpu.sync_copy(in1_hbm.at[pl.ds(_mo(base0 - 128, 128), 128)], abuf.at[pl.ds(0, 128)])
        pltpu.sync_copy(in2_hbm.at[pl.ds(_mo(base0 - 128, 128), 128)], bbuf.at[pl.ds(0, 128)])
    pa0 = jnp.sum(jnp.where(IOTA() == 15, abuf[pl.ds(112, 16)], 0), dtype=I32)
    pb0 = jnp.sum(jnp.where(IOTA() == 15, bbuf[pl.ds(112, 16)], 0), dtype=I32)

    def win(j, carry):
        prev_a, prev_b, open_c, nflag = carry
        off = base0 + j * WIN
        _stage_fwin(in1_hbm, in2_hbm, w_hbm, rs_hbm, abuf, bbuf, t2d, s2d,
                    w2d, rs2d, off, sem)

        def vr(v, vc):
            p_a, p_b, op, nf = vc
            rr = lax.div(v, I32(8))
            cc = lax.rem(v, I32(8)) * 16
            wv = w2d[rr, pl.ds(cc, 16)]
            rsv = rs2d[rr, pl.ds(cc, 16)]
            r = _flags_and_scan(abuf, bbuf, wv, rsv, shs, shb, v, p_a, p_b,
                                op, off)
            s15 = jnp.sum(jnp.where(IOTA() == 15, r["s"], 0.0))
            a15 = jnp.sum(jnp.where(IOTA() == 15, r["a"], 0), dtype=I32)
            b15 = jnp.sum(jnp.where(IOTA() == 15, r["b"], 0), dtype=I32)
            nf2 = nf + jnp.sum(r["flag"], dtype=I32)
            return (a15, b15, s15, nf2)
        p_a, p_b, op, nf = lax.fori_loop(I32(0), I32(WIN // 16), vr,
                                         (prev_a, prev_b, open_c, nflag))
        return (p_a, p_b, op, nf)

    _, _, open_f, nflag = lax.fori_loop(
        I32(0), cnt, win, (pa0, pb0, jnp.float32(0.0), I32(0)))

    mb[...] = jnp.where(IOTA() == 0, _full(nflag), 0)
    pltpu.sync_copy(mb, fc_hbm.at[pl.ds(_mo(wid * 16, 16), 16)])
    mbf[...] = jnp.where(IOTA() == 0, jnp.zeros((16,), F32) + open_f, 0.0)
    pltpu.sync_copy(mbf, ts_hbm.at[pl.ds(_mo(wid * 16, 16), 16)])


def _f1(in1, in2, w, rs):
    kern = functools.partial(
        pl.kernel,
        out_type=[jax.ShapeDtypeStruct((NW * 16,), I32),
                  jax.ShapeDtypeStruct((NW * 16,), F32)],
        mesh=_MESH, compiler_params=_CP,
        scratch_types=[
            pltpu.VMEM((WIN + 16,), I32),
            pltpu.VMEM((WIN + 16,), I32),
            pltpu.VMEM((WIN // 128, 128), I32),
            pltpu.VMEM((WIN // 128, 128), I32),
            pltpu.VMEM((WIN // 128, 128), F32),
            pltpu.VMEM((WIN // 128, 128), F32),
            pltpu.VMEM((16,), F32),
            pltpu.VMEM((16,), I32),
            pltpu.VMEM((16,), I32),
            pltpu.VMEM((16,), F32),
            pltpu.SemaphoreType.DMA,
        ],
    )(_f1_body)
    return kern(in1, in2, w, rs)


def _f2_body(in1_hbm, in2_hbm, w_hbm, rs_hbm, auxi_hbm, auxf_hbm,
             vals_hbm, srcO_hbm, dstO_hbm, bv_hbm, bs_hbm, bd_hbm,
             hv_hbm, hs_hbm, hd_hbm, hm_hbm, bm_hbm,
             abuf, bbuf, t2d, s2d, w2d, rs2d, shs, shb,
             ringv, rings, ringd, headv, heads, headd, hmk, mb, mbf, sem):
    wid = _wid()
    start, cnt = _win_range(wid, WIN_Q, WIN_R)
    base0 = start * WIN

    pltpu.sync_copy(auxi_hbm.at[pl.ds(_mo(wid * 16, 16), 16)], mb)
    sb = jnp.max(mb[...])
    pltpu.sync_copy(auxf_hbm.at[pl.ds(_mo(wid * 16, 16), 16)], mbf)
    inc0 = jnp.max(mbf[...])
    fb0 = (sb + 7) & (-8)
    hmk[...] = jnp.zeros((16,), I32)

    @pl.when(wid > 0)
    def _():
        pltpu.sync_copy(in1_hbm.at[pl.ds(_mo(base0 - 128, 128), 128)], abuf.at[pl.ds(0, 128)])
        pltpu.sync_copy(in2_hbm.at[pl.ds(_mo(base0 - 128, 128), 128)], bbuf.at[pl.ds(0, 128)])
    pa0 = jnp.sum(jnp.where(IOTA() == 15, abuf[pl.ds(112, 16)], 0), dtype=I32)
    pb0 = jnp.sum(jnp.where(IOTA() == 15, bbuf[pl.ds(112, 16)], 0), dtype=I32)

    def win(j, carry):
        prev_a, prev_b, open_c, seg_c, s1, fbase = carry
        off = base0 + j * WIN
        _stage_fwin(in1_hbm, in2_hbm, w_hbm, rs_hbm, abuf, bbuf, t2d, s2d,
                    w2d, rs2d, off, sem)

        def vr(v, vc):
            p_a, p_b, op, sc, ss1 = vc
            rr = lax.div(v, I32(8))
            cc = lax.rem(v, I32(8)) * 16
            wv = w2d[rr, pl.ds(cc, 16)]
            rsv = rs2d[rr, pl.ds(cc, 16)]
            r = _flags_and_scan(abuf, bbuf, wv, rsv, shs, shb, v, p_a, p_b,
                                op, off)
            seg = sc + r["fc"]
            is_tail = r["is_tail"]
            srcv = s2d[rr, pl.ds(cc, 16)]
            dstv = r["a"] & 0x1FFFF
            idx0 = seg & (RING - 1)
            plsc.store_scatter(ringv, [idx0], r["s"], mask=is_tail)
            plsc.store_scatter(rings, [idx0], srcv, mask=is_tail)
            plsc.store_scatter(ringd, [idx0], dstv, mask=is_tail)
            mwrap = is_tail & (idx0 < WIN)
            idx1 = jnp.where(idx0 < WIN, idx0 + RING, 0)
            plsc.store_scatter(ringv, [idx1], r["s"], mask=mwrap)
            plsc.store_scatter(rings, [idx1], srcv, mask=mwrap)
            plsc.store_scatter(ringd, [idx1], dstv, mask=mwrap)
            mh = is_tail & (seg < fb0)
            hix = jnp.clip(seg - sb, 0, 15)
            plsc.store_scatter(headv, [hix], r["s"], mask=mh)
            plsc.store_scatter(heads, [hix], srcv, mask=mh)
            plsc.store_scatter(headd, [hix], dstv, mask=mh)
            plsc.store_scatter(hmk, [hix], _full(1), mask=mh)
            s15 = jnp.sum(jnp.where(IOTA() == 15, r["s"], 0.0))
            a15 = jnp.sum(jnp.where(IOTA() == 15, r["a"], 0), dtype=I32)
            b15 = jnp.sum(jnp.where(IOTA() == 15, r["b"], 0), dtype=I32)
            sc2 = sc + jnp.sum(r["flag"], dtype=I32)
            ss1b = jnp.maximum(ss1, jnp.max(jnp.where(is_tail, seg, -1)))
            return (a15, b15, s15, sc2, ss1b)

        p_a, p_b, op, sc, ss1 = lax.fori_loop(
            I32(0), I32(WIN // 16), vr, (prev_a, prev_b, open_c, seg_c, s1))

        do_flush = fbase + (WIN - 1) <= ss1

        @pl.when(do_flush)
        def _():
            o = _mo(fbase & (RING - 1), 8)
            fb8 = _mo(fbase, 8)
            pltpu.sync_copy(ringv.at[pl.ds(o, WIN)], vals_hbm.at[pl.ds(fb8, WIN)])
            pltpu.sync_copy(rings.at[pl.ds(o, WIN)], srcO_hbm.at[pl.ds(fb8, WIN)])
            pltpu.sync_copy(ringd.at[pl.ds(o, WIN)], dstO_hbm.at[pl.ds(fb8, WIN)])
        fbase = jnp.where(do_flush, fbase + WIN, fbase)
        return (p_a, p_b, op, sc, ss1, fbase)

    init = (pa0, pb0, jnp.float32(0.0), sb, sb - 1, fb0)
    _, _, _, _, s1f, fbf = lax.fori_loop(I32(0), cnt, win, init)

    o = _mo(fbf & (RING - 1), 8)
    pltpu.sync_copy(ringv.at[pl.ds(o, WIN)], bv_hbm.at[pl.ds(_mo(wid * WIN, 128), WIN)])
    pltpu.sync_copy(rings.at[pl.ds(o, WIN)], bs_hbm.at[pl.ds(_mo(wid * WIN, 128), WIN)])
    pltpu.sync_copy(ringd.at[pl.ds(o, WIN)], bd_hbm.at[pl.ds(_mo(wid * WIN, 128), WIN)])
    pltpu.sync_copy(headv, hv_hbm.at[pl.ds(_mo(wid * 16, 16), 16)])
    pltpu.sync_copy(heads, hs_hbm.at[pl.ds(_mo(wid * 16, 16), 16)])
    pltpu.sync_copy(headd, hd_hbm.at[pl.ds(_mo(wid * 16, 16), 16)])
    pltpu.sync_copy(hmk, hm_hbm.at[pl.ds(_mo(wid * 16, 16), 16)])
    meta = jnp.where(IOTA() == 0, _full(fbf),
                     jnp.where(IOTA() == 1, _full(s1f + 1 - fbf),
                               jnp.where(IOTA() == 2, _full(sb),
                                         jnp.where(IOTA() == 3, _full(fb0 - sb),
                                                   _full(0)))))
    mb[...] = meta
    pltpu.sync_copy(mb, bm_hbm.at[pl.ds(_mo(wid * 16, 16), 16)])


def _f2(in1, in2, w, rs, aux_i, aux_f):
    kern = functools.partial(
        pl.kernel,
        out_type=[jax.ShapeDtypeStruct((N + WIN,), F32),
                  jax.ShapeDtypeStruct((N + WIN,), I32),
                  jax.ShapeDtypeStruct((N + WIN,), I32),
                  jax.ShapeDtypeStruct((NW * WIN,), F32),
                  jax.ShapeDtypeStruct((NW * WIN,), I32),
                  jax.ShapeDtypeStruct((NW * WIN,), I32),
                  jax.ShapeDtypeStruct((NW * 16,), F32),
                  jax.ShapeDtypeStruct((NW * 16,), I32),
                  jax.ShapeDtypeStruct((NW * 16,), I32),
                  jax.ShapeDtypeStruct((NW * 16,), I32),
                  jax.ShapeDtypeStruct((NW * 16,), I32)],
        mesh=_MESH, compiler_params=_CP,
        scratch_types=[
            pltpu.VMEM((WIN + 16,), I32),
            pltpu.VMEM((WIN + 16,), I32),
            pltpu.VMEM((WIN // 128, 128), I32),
            pltpu.VMEM((WIN // 128, 128), I32),
            pltpu.VMEM((WIN // 128, 128), F32),
            pltpu.VMEM((WIN // 128, 128), F32),
            pltpu.VMEM((16,), F32),
            pltpu.VMEM((16,), I32),
            pltpu.VMEM((RINGPAD,), F32),
            pltpu.VMEM((RINGPAD,), I32),
            pltpu.VMEM((RINGPAD,), I32),
            pltpu.VMEM((16,), F32),
            pltpu.VMEM((16,), I32),
            pltpu.VMEM((16,), I32),
            pltpu.VMEM((16,), I32),
            pltpu.VMEM((16,), I32),
            pltpu.VMEM((16,), F32),
            pltpu.SemaphoreType.DMA,
        ],
    )(_f2_body)
    return kern(in1, in2, w, rs, aux_i, aux_f)


def kernel(src, dst, emb1, emb2):
    src32 = src.astype(I32)
    dst32 = dst.astype(I32)
    e1pad = jnp.pad(emb1, ((0, 0), (0, 128 - EMB)))
    e2pad = jnp.pad(emb2, ((0, 0), (0, 128 - EMB)))

    w, rs2 = _k1(src32, dst32, e1pad, e2pad)
    rs = rs2[:V] + rs2[V:]

    a, b = src32, dst32
    for p in range(3):
        h = _hist(p, a, b)
        offs = _offsets(h, NB[p])
        a, b = _permute(p, a, b, offs)

    pad = jnp.zeros((128,), I32)
    a = jnp.concatenate([a, pad])
    b = jnp.concatenate([b, pad])

    fci, tsf = _f1(a, b, w, rs)
    fc = fci.reshape(NW, 16)[:, 0]
    ts = tsf.reshape(NW, 16)[:, 0]
    seg_base = jnp.concatenate(
        [jnp.zeros((1,), I32), jnp.cumsum(fc)[:-1].astype(I32)])
    U = (jnp.sum(fc) + 1).astype(I32)
    inc = []
    carry = jnp.float32(0.0)
    for t in range(NW):
        inc.append(carry)
        carry = ts[t] + jnp.where(fc[t] == 0, carry, jnp.float32(0.0))
    incoming = jnp.stack(inc).astype(F32)
    aux_i = jnp.broadcast_to(seg_base[:, None], (NW, 16)).astype(I32).reshape(-1)
    aux_f = jnp.broadcast_to(incoming[:, None], (NW, 16)).astype(F32).reshape(-1)

    valsR, srcR, dstR, bv, bs, bd, hv, hsv, hd, hm, bm = _f2(
        a, b, w, rs, aux_i, aux_f)
    hm = hm.reshape(NW, 16)
    bv = bv.reshape(NW, WIN)
    bs = bs.reshape(NW, WIN)
    bd = bd.reshape(NW, WIN)
    hv = hv.reshape(NW, 16)
    hsv = hsv.reshape(NW, 16)
    hd = hd.reshape(NW, 16)
    bm = bm.reshape(NW, 16)

    oob = I32(N + WIN + 7)
    ar = jnp.arange(WIN, dtype=I32)
    tpos = bm[:, 0:1] + ar[None, :]
    tmask = ar[None, :] < bm[:, 1:2]
    tgt = jnp.where(tmask, tpos, oob).reshape(-1)
    valsR = valsR.at[tgt].set(bv.reshape(-1), mode="drop")
    srcR = srcR.at[tgt].set(bs.reshape(-1), mode="drop")
    dstR = dstR.at[tgt].set(bd.reshape(-1), mode="drop")

    ar16 = jnp.arange(16, dtype=I32)
    hpos = bm[:, 2:3] + ar16[None, :]
    hmask = (ar16[None, :] < bm[:, 3:4]) & (hm > 0)
    htgt = jnp.where(hmask, hpos, oob).reshape(-1)
    valsR = valsR.at[htgt].set(hv.reshape(-1), mode="drop")
    srcR = srcR.at[htgt].set(hsv.reshape(-1), mode="drop")
    dstR = dstR.at[htgt].set(hd.reshape(-1), mode="drop")

    ii = jnp.arange(N, dtype=I32)
    ok = ii < U
    vals = jnp.where(ok, valsR[:N], jnp.float32(0.0)).astype(F32)
    srcO = jnp.where(ok, srcR[:N], 0)
    dstO = jnp.where(ok, dstR[:N], 0)
    idx = jnp.stack([srcO.astype(jnp.int64), dstO.astype(jnp.int64)], axis=0)
    return idx, vals


# speedup vs baseline: 52.1739x; 1.1018x over previous
import functools

import numpy as np
import jax
import jax.numpy as jnp
from jax import lax
from jax.experimental import pallas as pl
from jax.experimental.pallas import tpu as pltpu
from jax.experimental.pallas import tpu_sc as plsc

N = 3200000
V = 100000
EMB = 16
EPS = 1e-08

NC = 2
NS = 16
NW = NC * NS

I32 = jnp.int32
F32 = jnp.float32

WIN = 1024
NWIN = N // WIN
WIN_Q, WIN_R = divmod(NWIN, NW)

KWIN = 512
KNWIN = N // KWIN
KWIN_Q, KWIN_R = divmod(KNWIN, NW)

SSEG = 800000
NROUND = 4
SSUB = SSEG // NS

NB = [2048, 4096, 2048]

RING = 4096
RINGPAD = RING + WIN

_MESH = plsc.VectorSubcoreMesh(core_axis_name="c", subcore_axis_name="s")
_CP = pltpu.CompilerParams(needs_layout_passes=False)

IOTA = lambda: lax.iota(I32, 16)


def _mo(x, m):
    return pl.multiple_of(x, m)


def _wid():
    return lax.axis_index("s") * NC + lax.axis_index("c")


def _full(v):
    return jnp.zeros((16,), I32) + v


def _win_range(wid, q, r):
    start = wid * q + jnp.minimum(wid, r)
    cnt = jnp.where(wid < r, I32(q + 1), I32(q))
    return start.astype(I32), cnt


def _splat_lane(ref16, lane):
    return plsc.load_gather(ref16, [_full(lane)])


def _srl(x, k):
    return lax.shift_right_logical(x, jnp.full(x.shape, k, I32))


def _digit(p, a, b):
    if p == 0:
        return a & 0x7FF
    if p == 1:
        return ((_srl(a, 17) & 0x3F) << 6) | (_srl(a, 11) & 0x3F)
    return ((_srl(b, 22) & 0x3) << 9) | (_srl(a, 23) & 0x1FF)


def _decode_src(a, b):
    return (_srl(a, 17) & 0x7FFF) | ((_srl(b, 22) & 0x3) << 15)


def _k1_body(src_hbm, dst_hbm, e1_hbm, e2_hbm, w_hbm, rs_hbm,
             s2d, d2d, r1, r2, w2d, zb, rs_sh, sem):
    wid = _wid()
    cid = lax.axis_index("c")
    sid = lax.axis_index("s")
    nz = 6256 // 16
    def zinit(i, _):
        zb[pl.ds(i * 16, 16)] = jnp.zeros((16,), F32)
        return 0
    lax.fori_loop(I32(0), I32(nz), zinit, 0)

    @pl.when(sid < 15)
    def _():
        pltpu.sync_copy(zb.at[pl.ds(0, 6256)],
                        rs_sh.at[pl.ds(_mo(sid * 6256, 16), 6256)])

    @pl.when(sid == 15)
    def _():
        pltpu.sync_copy(zb.at[pl.ds(0, 6160)],
                        rs_sh.at[pl.ds(_mo(sid * 6256, 16), 6160)])
    plsc.subcore_barrier()

    start, cnt = _win_range(wid, KWIN_Q, KWIN_R)
    nslice = KWIN // 128

    def win(j, _):
        off = _mo((start + j) * KWIN, 128)
        for k in range(nslice):
            pltpu.sync_copy(src_hbm.at[pl.ds(_mo(off + k * 128, 128), 128)], s2d.at[I32(k)])
            pltpu.sync_copy(dst_hbm.at[pl.ds(_mo(off + k * 128, 128), 128)], d2d.at[I32(k)])

        hs0 = [(pltpu.async_copy(e1_hbm.at[s2d.at[I32(q)]], r1.at[I32(q & 1)], sem),
                pltpu.async_copy(e2_hbm.at[d2d.at[I32(q)]], r2.at[I32(q & 1)], sem))
               if q < 1 else None for q in range(1)]
        for q in range(nslice):
            if q == 0:
                h1, h2 = hs0[0]
            h1.wait()
            h2.wait()
            if q + 1 < nslice:
                h1 = pltpu.async_copy(e1_hbm.at[s2d.at[I32(q + 1)]],
                                      r1.at[I32((q + 1) & 1)], sem)
                h2 = pltpu.async_copy(e2_hbm.at[d2d.at[I32(q + 1)]],
                                      r2.at[I32((q + 1) & 1)], sem)
            r1q = r1.at[I32(q & 1)]
            r2q = r2.at[I32(q & 1)]

            def dot(v, _):
                acc = jnp.zeros((16,), F32)
                ev = v * 16 + IOTA()
                for d in range(16):
                    dd = _full(d)
                    acc = acc + (plsc.load_gather(r1q, [ev, dd])
                                 * plsc.load_gather(r2q, [ev, dd]))
                w = 1.0 / (1.0 + jnp.exp(-acc))
                w2d[I32(q), pl.ds(v * 16, 16)] = w
                return 0
            lax.fori_loop(I32(0), I32(8), dot, 0)

        for k in range(nslice):
            pltpu.sync_copy(w2d.at[I32(k)], w_hbm.at[pl.ds(_mo(off + k * 128, 128), 128)])
        hs = [pltpu.async_copy(w2d.at[I32(k)], rs_sh.at[s2d.at[I32(k)]], sem, add=True)
              for k in range(nslice)]
        for h in hs:
            h.wait()
        return 0

    lax.fori_loop(I32(0), cnt, win, 0)
    plsc.subcore_barrier()

    @pl.when(sid < 15)
    def _():
        pltpu.sync_copy(rs_sh.at[pl.ds(_mo(sid * 6256, 16), 6256)],
                        zb.at[pl.ds(0, 6256)])
        pltpu.sync_copy(zb.at[pl.ds(0, 6256)],
                        rs_hbm.at[pl.ds(_mo(cid * V + sid * 6256, 16), 6256)])

    @pl.when(sid == 15)
    def _():
        pltpu.sync_copy(rs_sh.at[pl.ds(_mo(sid * 6256, 16), 6160)],
                        zb.at[pl.ds(0, 6160)])
        pltpu.sync_copy(zb.at[pl.ds(0, 6160)],
                        rs_hbm.at[pl.ds(_mo(cid * V + sid * 6256, 16), 6160)])


def _k1(src32, dst32, e1pad, e2pad):
    kern = functools.partial(
        pl.kernel,
        out_type=[jax.ShapeDtypeStruct((N,), F32),
                  jax.ShapeDtypeStruct((NC * V,), F32)],
        mesh=_MESH, compiler_params=_CP,
        scratch_types=[
            pltpu.VMEM((KWIN // 128, 128), I32),
            pltpu.VMEM((KWIN // 128, 128), I32),
            pltpu.VMEM((2, 128, 128), F32),
            pltpu.VMEM((2, 128, 128), F32),
            pltpu.VMEM((KWIN // 128, 128), F32),
            pltpu.VMEM((6256,), F32),
            pltpu.VMEM_SHARED((V,), F32),
            pltpu.SemaphoreType.DMA,
        ],
    )(_k1_body)
    return kern(src32, dst32, e1pad, e2pad)


def _h_body(p, nb, in1_hbm, in2_hbm, hist_hbm, abuf, bbuf, hist):
    wid = _wid()
    def zi(i, _):
        hist[pl.ds(i * 16, 16)] = jnp.zeros((16,), I32)
        return 0
    lax.fori_loop(I32(0), I32(nb // 16), zi, 0)

    start, cnt = _win_range(wid, WIN_Q, WIN_R)

    def win(j, _):
        off = _mo((start + j) * WIN, 128)
        pltpu.sync_copy(in1_hbm.at[pl.ds(off, WIN)], abuf)
        pltpu.sync_copy(in2_hbm.at[pl.ds(off, WIN)], bbuf)

        def vr(v, _):
            a = abuf[pl.ds(v * 16, 16)]
            b = bbuf[pl.ds(v * 16, 16)]
            if p == 0:
                d = b & 0x7FF
            else:
                d = _digit(p, a, b)
            cntv, lastm = plsc.scan_count(d)
            base = plsc.load_gather(hist, [d])
            plsc.store_scatter(hist, [d], base + cntv, mask=lastm)
            return 0
        lax.fori_loop(I32(0), I32(WIN // 16), vr, 0)
        return 0

    lax.fori_loop(I32(0), cnt, win, 0)
    pltpu.sync_copy(hist, hist_hbm.at[pl.ds(_mo(wid * nb, 128), nb)])


def _hist(p, in1, in2):
    nb = NB[p]
    body = functools.partial(_h_body, p, nb)
    kern = functools.partial(
        pl.kernel,
        out_type=jax.ShapeDtypeStruct((NW * nb,), I32),
        mesh=_MESH, compiler_params=_CP,
        scratch_types=[
            pltpu.VMEM((WIN,), I32),
            pltpu.VMEM((WIN,), I32),
            pltpu.VMEM((nb,), I32),
        ],
    )(body)
    return kern(in1, in2)


def _offsets(hist, nb):
    hist = hist.reshape(NW, nb)
    flat = hist.T.reshape(-1).astype(I32)
    ex = jnp.concatenate([jnp.zeros((1,), I32), jnp.cumsum(flat)[:-1].astype(I32)])
    return ex.reshape(nb, NW).T.reshape(-1)


def _p_body(p, nb, in1_hbm, in2_hbm, offs_hbm, a_out, b_out,
            abuf, bbuf, av2d, bv2d, pos2d, counter, zb, bounce,
            segA, segB, sem):
    wid = _wid()
    cid = lax.axis_index("c")
    sid = lax.axis_index("s")
    start, cnt = _win_range(wid, WIN_Q, WIN_R)
    nzc = SSUB // 2000

    def zvi(i, _):
        zb[pl.ds(i * 16, 16)] = jnp.zeros((16,), I32)
        return 0
    lax.fori_loop(I32(0), I32(2000 // 16), zvi, 0)

    for r in range(NROUND):
        def zc(i, _):
            pltpu.sync_copy(zb, segA.at[pl.ds(_mo(sid * SSUB + i * 2000, 8), 2000)])
            pltpu.sync_copy(zb, segB.at[pl.ds(_mo(sid * SSUB + i * 2000, 8), 2000)])
            return 0
        lax.fori_loop(I32(0), I32(nzc), zc, 0)
        plsc.subcore_barrier()

        pltpu.sync_copy(offs_hbm.at[pl.ds(_mo(wid * nb, 128), nb)], counter)
        lo = r * SSEG

        def win(j, _):
            off = _mo((start + j) * WIN, 128)
            pltpu.sync_copy(in1_hbm.at[pl.ds(off, WIN)], abuf)
            pltpu.sync_copy(in2_hbm.at[pl.ds(off, WIN)], bbuf)

            def vr(v, _):
                x1 = abuf[pl.ds(v * 16, 16)]
                x2 = bbuf[pl.ds(v * 16, 16)]
                if p == 0:
                    tag = off + v * 16 + IOTA()
                    a = x2 | ((x1 & 0x7FFF) << 17)
                    b = tag | (_srl(x1, 15) << 22)
                    d = x2 & 0x7FF
                else:
                    a, b = x1, x2
                    d = _digit(p, a, b)
                cntv, lastm = plsc.scan_count(d)
                base = plsc.load_gather(counter, [d])
                plsc.store_scatter(counter, [d], base + cntv, mask=lastm)
                pos = base + cntv - 1
                m = (pos >= lo) & (pos < lo + SSEG)
                iv = jnp.where(m, pos - lo, SSEG + IOTA())
                rr = lax.div(v, I32(8))
                cc = lax.rem(v, I32(8)) * 16
                pos2d[rr, pl.ds(cc, 16)] = iv
                av2d[rr, pl.ds(cc, 16)] = a
                bv2d[rr, pl.ds(cc, 16)] = b
                return 0
            lax.fori_loop(I32(0), I32(WIN // 16), vr, 0)

            hs = []
            for k in range(WIN // 128):
                hs.append(pltpu.async_copy(av2d.at[I32(k)], segA.at[pos2d.at[I32(k)]], sem))
                hs.append(pltpu.async_copy(bv2d.at[I32(k)], segB.at[pos2d.at[I32(k)]], sem))
            for h in hs:
                h.wait()
            return 0

        lax.fori_loop(I32(0), cnt, win, 0)
        plsc.subcore_barrier()

        def ec(i, _):
            o = _mo(sid * SSUB + i * 2000, 8)
            pltpu.sync_copy(segA.at[pl.ds(o, 2000)], bounce)
            pltpu.sync_copy(bounce, a_out.at[pl.ds(_mo(cid * N + lo + o, 8), 2000)])
            pltpu.sync_copy(segB.at[pl.ds(o, 2000)], bounce)
            pltpu.sync_copy(bounce, b_out.at[pl.ds(_mo(cid * N + lo + o, 8), 2000)])
            return 0
        lax.fori_loop(I32(0), I32(nzc), ec, 0)
        plsc.subcore_barrier()


def _permute(p, in1, in2, offs):
    nb = NB[p]
    body = functools.partial(_p_body, p, nb)
    kern = functools.partial(
        pl.kernel,
        out_type=[jax.ShapeDtypeStruct((NC * N,), I32),
                  jax.ShapeDtypeStruct((NC * N,), I32)],
        mesh=_MESH, compiler_params=_CP,
        scratch_types=[
            pltpu.VMEM((WIN,), I32),
            pltpu.VMEM((WIN,), I32),
            pltpu.VMEM((WIN // 128, 128), I32),
            pltpu.VMEM((WIN // 128, 128), I32),
            pltpu.VMEM((WIN // 128, 128), I32),
            pltpu.VMEM((nb,), I32),
            pltpu.VMEM((2000,), I32),
            pltpu.VMEM((2000,), I32),
            pltpu.VMEM_SHARED((SSEG + 16,), I32),
            pltpu.VMEM_SHARED((SSEG + 16,), I32),
            pltpu.SemaphoreType.DMA,
        ],
    )(body)
    ao, bo = kern(in1, in2, offs)
    return ao[:N] + ao[N:], bo[:N] + bo[N:]


def _flags_and_scan(abuf, bbuf, wv, rsv, shs, shb, v, prev_a, prev_b,
                    open_c, base_off):
    el = v * 16 + IOTA()
    a = abuf[pl.ds(v * 16, 16)]
    b = bbuf[pl.ds(v * 16, 16)]
    elm1 = jnp.maximum(el - 1, 0)
    ap = plsc.load_gather(abuf, [elm1])
    bp = plsc.load_gather(bbuf, [elm1])
    first = el == 0
    ap = jnp.where(first, _full(prev_a), ap)
    bp = jnp.where(first, _full(prev_b), bp)
    ge = base_off + el
    flag = ((a != ap) | (_srl(b, 22) != _srl(bp, 22))).astype(I32)
    flag = jnp.where(ge == 0, 0, flag)

    wn = wv / (rsv + EPS)

    fc = plsc.cumsum(flag)
    s = wn
    bb = flag
    for k in (1, 2, 4, 8):
        shs[...] = s
        shb[...] = bb
        gidx = jnp.maximum(IOTA() - k, 0)
        shv = plsc.load_gather(shs, [gidx])
        bhv = plsc.load_gather(shb, [gidx])
        ok = IOTA() >= k
        s = s + jnp.where(ok & (bb == 0), shv, 0.0)
        bb = bb | jnp.where(ok, bhv, 0)
    s = s + jnp.where(fc == 0, open_c, 0.0)

    an = plsc.load_gather(abuf, [el + 1])
    bn = plsc.load_gather(bbuf, [el + 1])
    flagn = ((a != an) | (_srl(b, 22) != _srl(bn, 22)))
    is_tail = flagn | (ge == N - 1)
    return dict(a=a, b=b, el=el, flag=flag, fc=fc, s=s, is_tail=is_tail)


def _stage_fwin(in1_hbm, in2_hbm, w_hbm, rs_hbm, abuf, bbuf, t2d, s2d,
                w2d, rs2d, off, sem):
    pltpu.sync_copy(in1_hbm.at[pl.ds(_mo(off, 128), WIN + 16)], abuf)
    pltpu.sync_copy(in2_hbm.at[pl.ds(_mo(off, 128), WIN + 16)], bbuf)

    def dv(v, _):
        a = abuf[pl.ds(v * 16, 16)]
        b = bbuf[pl.ds(v * 16, 16)]
        rr = lax.div(v, I32(8))
        cc = lax.rem(v, I32(8)) * 16
        t2d[rr, pl.ds(cc, 16)] = b & 0x3FFFFF
        s2d[rr, pl.ds(cc, 16)] = _decode_src(a, b)
        return 0
    lax.fori_loop(I32(0), I32(WIN // 16), dv, 0)
    hs = []
    for k in range(WIN // 128):
        hs.append(pltpu.async_copy(w_hbm.at[t2d.at[I32(k)]], w2d.at[I32(k)], sem))
        hs.append(pltpu.async_copy(rs_hbm.at[s2d.at[I32(k)]], rs2d.at[I32(k)], sem))
    for h in hs:
        h.wait()


def _f1_body(in1_hbm, in2_hbm, w_hbm, rs_hbm, fc_hbm, ts_hbm,
             abuf, bbuf, t2d, s2d, w2d, rs2d, shs, shb, mb, mbf, sem):
    wid = _wid()
    start, cnt = _win_range(wid, WIN_Q, WIN_R)
    base0 = start * WIN

    @pl.when(wid > 0)
    def _():
        pltpu.sync_copy(in1_hbm.at[pl.ds(_mo(base0 - 128, 128), 128)], abuf.at[pl.ds(0, 128)])
        pltpu.sync_copy(in2_hbm.at[pl.ds(_mo(base0 - 128, 128), 128)], bbuf.at[pl.ds(0, 128)])
    pa0 = jnp.sum(jnp.where(IOTA() == 15, abuf[pl.ds(112, 16)], 0), dtype=I32)
    pb0 = jnp.sum(jnp.where(IOTA() == 15, bbuf[pl.ds(112, 16)], 0), dtype=I32)

    def win(j, carry):
        prev_a, prev_b, open_c, nflag = carry
        off = base0 + j * WIN
        _stage_fwin(in1_hbm, in2_hbm, w_hbm, rs_hbm, abuf, bbuf, t2d, s2d,
                    w2d, rs2d, off, sem)

        def vr(v, vc):
            p_a, p_b, op, nf = vc
            rr = lax.div(v, I32(8))
            cc = lax.rem(v, I32(8)) * 16
            wv = w2d[rr, pl.ds(cc, 16)]
            rsv = rs2d[rr, pl.ds(cc, 16)]
            r = _flags_and_scan(abuf, bbuf, wv, rsv, shs, shb, v, p_a, p_b,
                                op, off)
            s15 = jnp.sum(jnp.where(IOTA() == 15, r["s"], 0.0))
            a15 = jnp.sum(jnp.where(IOTA() == 15, r["a"], 0), dtype=I32)
            b15 = jnp.sum(jnp.where(IOTA() == 15, r["b"], 0), dtype=I32)
            nf2 = nf + jnp.sum(r["flag"], dtype=I32)
            return (a15, b15, s15, nf2)
        p_a, p_b, op, nf = lax.fori_loop(I32(0), I32(WIN // 16), vr,
                                         (prev_a, prev_b, open_c, nflag))
        return (p_a, p_b, op, nf)

    _, _, open_f, nflag = lax.fori_loop(
        I32(0), cnt, win, (pa0, pb0, jnp.float32(0.0), I32(0)))

    mb[...] = jnp.where(IOTA() == 0, _full(nflag), 0)
    pltpu.sync_copy(mb, fc_hbm.at[pl.ds(_mo(wid * 16, 16), 16)])
    mbf[...] = jnp.where(IOTA() == 0, jnp.zeros((16,), F32) + open_f, 0.0)
    pltpu.sync_copy(mbf, ts_hbm.at[pl.ds(_mo(wid * 16, 16), 16)])


def _f1(in1, in2, w, rs):
    kern = functools.partial(
        pl.kernel,
        out_type=[jax.ShapeDtypeStruct((NW * 16,), I32),
                  jax.ShapeDtypeStruct((NW * 16,), F32)],
        mesh=_MESH, compiler_params=_CP,
        scratch_types=[
            pltpu.VMEM((WIN + 16,), I32),
            pltpu.VMEM((WIN + 16,), I32),
            pltpu.VMEM((WIN // 128, 128), I32),
            pltpu.VMEM((WIN // 128, 128), I32),
            pltpu.VMEM((WIN // 128, 128), F32),
            pltpu.VMEM((WIN // 128, 128), F32),
            pltpu.VMEM((16,), F32),
            pltpu.VMEM((16,), I32),
            pltpu.VMEM((16,), I32),
            pltpu.VMEM((16,), F32),
            pltpu.SemaphoreType.DMA,
        ],
    )(_f1_body)
    return kern(in1, in2, w, rs)


def _f2_body(in1_hbm, in2_hbm, w_hbm, rs_hbm, auxi_hbm, auxf_hbm,
             vals_hbm, srcO_hbm, dstO_hbm, bv_hbm, bs_hbm, bd_hbm,
             hv_hbm, hs_hbm, hd_hbm, hm_hbm, bm_hbm,
             abuf, bbuf, t2d, s2d, w2d, rs2d, shs, shb,
             ringv, rings, ringd, headv, heads, headd, hmk, mb, mbf, sem):
    wid = _wid()
    start, cnt = _win_range(wid, WIN_Q, WIN_R)
    base0 = start * WIN

    pltpu.sync_copy(auxi_hbm.at[pl.ds(_mo(wid * 16, 16), 16)], mb)
    sb = jnp.max(mb[...])
    pltpu.sync_copy(auxf_hbm.at[pl.ds(_mo(wid * 16, 16), 16)], mbf)
    inc0 = jnp.max(mbf[...])
    fb0 = (sb + 7) & (-8)
    hmk[...] = jnp.zeros((16,), I32)

    @pl.when(wid > 0)
    def _():
        pltpu.sync_copy(in1_hbm.at[pl.ds(_mo(base0 - 128, 128), 128)], abuf.at[pl.ds(0, 128)])
        pltpu.sync_copy(in2_hbm.at[pl.ds(_mo(base0 - 128, 128), 128)], bbuf.at[pl.ds(0, 128)])
    pa0 = jnp.sum(jnp.where(IOTA() == 15, abuf[pl.ds(112, 16)], 0), dtype=I32)
    pb0 = jnp.sum(jnp.where(IOTA() == 15, bbuf[pl.ds(112, 16)], 0), dtype=I32)

    def win(j, carry):
        prev_a, prev_b, open_c, seg_c, s1, fbase = carry
        off = base0 + j * WIN
        _stage_fwin(in1_hbm, in2_hbm, w_hbm, rs_hbm, abuf, bbuf, t2d, s2d,
                    w2d, rs2d, off, sem)

        def vr(v, vc):
            p_a, p_b, op, sc, ss1 = vc
            rr = lax.div(v, I32(8))
            cc = lax.rem(v, I32(8)) * 16
            wv = w2d[rr, pl.ds(cc, 16)]
            rsv = rs2d[rr, pl.ds(cc, 16)]
            r = _flags_and_scan(abuf, bbuf, wv, rsv, shs, shb, v, p_a, p_b,
                                op, off)
            seg = sc + r["fc"]
            is_tail = r["is_tail"]
            srcv = s2d[rr, pl.ds(cc, 16)]
            dstv = r["a"] & 0x1FFFF
            idx0 = seg & (RING - 1)
            plsc.store_scatter(ringv, [idx0], r["s"], mask=is_tail)
            plsc.store_scatter(rings, [idx0], srcv, mask=is_tail)
            plsc.store_scatter(ringd, [idx0], dstv, mask=is_tail)
            mwrap = is_tail & (idx0 < WIN)
            idx1 = jnp.where(idx0 < WIN, idx0 + RING, 0)
            plsc.store_scatter(ringv, [idx1], r["s"], mask=mwrap)
            plsc.store_scatter(rings, [idx1], srcv, mask=mwrap)
            plsc.store_scatter(ringd, [idx1], dstv, mask=mwrap)
            mh = is_tail & (seg < fb0)
            hix = jnp.clip(seg - sb, 0, 15)
            plsc.store_scatter(headv, [hix], r["s"], mask=mh)
            plsc.store_scatter(heads, [hix], srcv, mask=mh)
            plsc.store_scatter(headd, [hix], dstv, mask=mh)
            plsc.store_scatter(hmk, [hix], _full(1), mask=mh)
            s15 = jnp.sum(jnp.where(IOTA() == 15, r["s"], 0.0))
            a15 = jnp.sum(jnp.where(IOTA() == 15, r["a"], 0), dtype=I32)
            b15 = jnp.sum(jnp.where(IOTA() == 15, r["b"], 0), dtype=I32)
            sc2 = sc + jnp.sum(r["flag"], dtype=I32)
            ss1b = jnp.maximum(ss1, jnp.max(jnp.where(is_tail, seg, -1)))
            return (a15, b15, s15, sc2, ss1b)

        p_a, p_b, op, sc, ss1 = lax.fori_loop(
            I32(0), I32(WIN // 16), vr, (prev_a, prev_b, open_c, seg_c, s1))

        do_flush = fbase + (WIN - 1) <= ss1

        @pl.when(do_flush)
        def _():
            o = _mo(fbase & (RING - 1), 8)
            fb8 = _mo(fbase, 8)
            pltpu.sync_copy(ringv.at[pl.ds(o, WIN)], vals_hbm.at[pl.ds(fb8, WIN)])
            pltpu.sync_copy(rings.at[pl.ds(o, WIN)], srcO_hbm.at[pl.ds(fb8, WIN)])
            pltpu.sync_copy(ringd.at[pl.ds(o, WIN)], dstO_hbm.at[pl.ds(fb8, WIN)])
        fbase = jnp.where(do_flush, fbase + WIN, fbase)
        return (p_a, p_b, op, sc, ss1, fbase)

    init = (pa0, pb0, jnp.float32(0.0), sb, sb - 1, fb0)
    _, _, _, _, s1f, fbf = lax.fori_loop(I32(0), cnt, win, init)

    o = _mo(fbf & (RING - 1), 8)
    pltpu.sync_copy(ringv.at[pl.ds(o, WIN)], bv_hbm.at[pl.ds(_mo(wid * WIN, 128), WIN)])
    pltpu.sync_copy(rings.at[pl.ds(o, WIN)], bs_hbm.at[pl.ds(_mo(wid * WIN, 128), WIN)])
    pltpu.sync_copy(ringd.at[pl.ds(o, WIN)], bd_hbm.at[pl.ds(_mo(wid * WIN, 128), WIN)])
    pltpu.sync_copy(headv, hv_hbm.at[pl.ds(_mo(wid * 16, 16), 16)])
    pltpu.sync_copy(heads, hs_hbm.at[pl.ds(_mo(wid * 16, 16), 16)])
    pltpu.sync_copy(headd, hd_hbm.at[pl.ds(_mo(wid * 16, 16), 16)])
    pltpu.sync_copy(hmk, hm_hbm.at[pl.ds(_mo(wid * 16, 16), 16)])
    meta = jnp.where(IOTA() == 0, _full(fbf),
                     jnp.where(IOTA() == 1, _full(s1f + 1 - fbf),
                               jnp.where(IOTA() == 2, _full(sb),
                                         jnp.where(IOTA() == 3, _full(fb0 - sb),
                                                   _full(0)))))
    mb[...] = meta
    pltpu.sync_copy(mb, bm_hbm.at[pl.ds(_mo(wid * 16, 16), 16)])


def _f2(in1, in2, w, rs, aux_i, aux_f):
    kern = functools.partial(
        pl.kernel,
        out_type=[jax.ShapeDtypeStruct((N + WIN,), F32),
                  jax.ShapeDtypeStruct((N + WIN,), I32),
                  jax.ShapeDtypeStruct((N + WIN,), I32),
                  jax.ShapeDtypeStruct((NW * WIN,), F32),
                  jax.ShapeDtypeStruct((NW * WIN,), I32),
                  jax.ShapeDtypeStruct((NW * WIN,), I32),
                  jax.ShapeDtypeStruct((NW * 16,), F32),
                  jax.ShapeDtypeStruct((NW * 16,), I32),
                  jax.ShapeDtypeStruct((NW * 16,), I32),
                  jax.ShapeDtypeStruct((NW * 16,), I32),
                  jax.ShapeDtypeStruct((NW * 16,), I32)],
        mesh=_MESH, compiler_params=_CP,
        scratch_types=[
            pltpu.VMEM((WIN + 16,), I32),
            pltpu.VMEM((WIN + 16,), I32),
            pltpu.VMEM((WIN // 128, 128), I32),
            pltpu.VMEM((WIN // 128, 128), I32),
            pltpu.VMEM((WIN // 128, 128), F32),
            pltpu.VMEM((WIN // 128, 128), F32),
            pltpu.VMEM((16,), F32),
            pltpu.VMEM((16,), I32),
            pltpu.VMEM((RINGPAD,), F32),
            pltpu.VMEM((RINGPAD,), I32),
            pltpu.VMEM((RINGPAD,), I32),
            pltpu.VMEM((16,), F32),
            pltpu.VMEM((16,), I32),
            pltpu.VMEM((16,), I32),
            pltpu.VMEM((16,), I32),
            pltpu.VMEM((16,), I32),
            pltpu.VMEM((16,), F32),
            pltpu.SemaphoreType.DMA,
        ],
    )(_f2_body)
    return kern(in1, in2, w, rs, aux_i, aux_f)


def kernel(src, dst, emb1, emb2):
    src32 = src.astype(I32)
    dst32 = dst.astype(I32)
    e1pad = jnp.pad(emb1, ((0, 0), (0, 128 - EMB)))
    e2pad = jnp.pad(emb2, ((0, 0), (0, 128 - EMB)))

    w, rs2 = _k1(src32, dst32, e1pad, e2pad)
    rs = rs2[:V] + rs2[V:]

    a, b = src32, dst32
    for p in range(3):
        h = _hist(p, a, b)
        offs = _offsets(h, NB[p])
        a, b = _permute(p, a, b, offs)

    pad = jnp.zeros((128,), I32)
    a = jnp.concatenate([a, pad])
    b = jnp.concatenate([b, pad])

    fci, tsf = _f1(a, b, w, rs)
    fc = fci.reshape(NW, 16)[:, 0]
    ts = tsf.reshape(NW, 16)[:, 0]
    seg_base = jnp.concatenate(
        [jnp.zeros((1,), I32), jnp.cumsum(fc)[:-1].astype(I32)])
    U = (jnp.sum(fc) + 1).astype(I32)
    inc = []
    carry = jnp.float32(0.0)
    for t in range(NW):
        inc.append(carry)
        carry = ts[t] + jnp.where(fc[t] == 0, carry, jnp.float32(0.0))
    incoming = jnp.stack(inc).astype(F32)
    aux_i = jnp.broadcast_to(seg_base[:, None], (NW, 16)).astype(I32).reshape(-1)
    aux_f = jnp.broadcast_to(incoming[:, None], (NW, 16)).astype(F32).reshape(-1)

    valsR, srcR, dstR, bv, bs, bd, hv, hsv, hd, hm, bm = _f2(
        a, b, w, rs, aux_i, aux_f)
    hm = hm.reshape(NW, 16)
    bv = bv.reshape(NW, WIN)
    bs = bs.reshape(NW, WIN)
    bd = bd.reshape(NW, WIN)
    hv = hv.reshape(NW, 16)
    hsv = hsv.reshape(NW, 16)
    hd = hd.reshape(NW, 16)
    bm = bm.reshape(NW, 16)

    oob = I32(N + WIN + 7)
    ar = jnp.arange(WIN, dtype=I32)
    tpos = bm[:, 0:1] + ar[None, :]
    tmask = ar[None, :] < bm[:, 1:2]
    tgt = jnp.where(tmask, tpos, oob).reshape(-1)
    valsR = valsR.at[tgt].set(bv.reshape(-1), mode="drop")
    srcR = srcR.at[tgt].set(bs.reshape(-1), mode="drop")
    dstR = dstR.at[tgt].set(bd.reshape(-1), mode="drop")

    ar16 = jnp.arange(16, dtype=I32)
    hpos = bm[:, 2:3] + ar16[None, :]
    hmask = (ar16[None, :] < bm[:, 3:4]) & (hm > 0)
    htgt = jnp.where(hmask, hpos, oob).reshape(-1)
    valsR = valsR.at[htgt].set(hv.reshape(-1), mode="drop")
    srcR = srcR.at[htgt].set(hsv.reshape(-1), mode="drop")
    dstR = dstR.at[htgt].set(hd.reshape(-1), mode="drop")

    ii = jnp.arange(N, dtype=I32)
    ok = ii < U
    vals = jnp.where(ok, valsR[:N], jnp.float32(0.0)).astype(F32)
    srcO = jnp.where(ok, srcR[:N], 0)
    dstO = jnp.where(ok, dstR[:N], 0)
    idx = jnp.stack([srcO.astype(jnp.int64), dstO.astype(jnp.int64)], axis=0)
    return idx, vals
